# baseline (device time: 40603 ns/iter reference)
import jax
import jax.numpy as jnp
from jax import lax
from jax.experimental import pallas as pl
from jax.experimental.pallas import tpu as pltpu

N_DEV = 32
N_LAYERS = 3
GROUPS = ((1, 3, 2, 4, 5, 7, 6), (8, 16, 24))
ALL_GENS = tuple(g for grp in GROUPS for g in grp)
N_STAGES = len(GROUPS)
N_RECV = len(ALL_GENS)
N_SLOTS = N_LAYERS * N_RECV


def kernel(x, Win0, Wout0, Win1, Wout1, Win2, Wout2):
    b, d = x.shape

    def body(x_ref, win0_ref, wout0_ref, win1_ref, wout1_ref, win2_ref,
             wout2_ref, out_ref, wbf_ref, obf_ref, send_ref, recv_ref,
             send_sems, recv_sems):
        my = lax.axis_index("i")

        barrier_sem = pltpu.get_barrier_semaphore()
        for g in ALL_GENS:
            pl.semaphore_signal(
                barrier_sem, inc=1,
                device_id=(my ^ g,), device_id_type=pl.DeviceIdType.MESH,
            )
        for i, (w, o) in enumerate(((win0_ref, wout0_ref),
                                    (win1_ref, wout1_ref),
                                    (win2_ref, wout2_ref))):
            wbf_ref[i] = w[...].astype(jnp.bfloat16)
            obf_ref[i] = o[...].astype(jnp.bfloat16)
        pl.semaphore_wait(barrier_sem, N_RECV)

        wins = tuple(wbf_ref.at[i] for i in range(N_LAYERS))
        wouts = tuple(obf_ref.at[i] for i in range(N_LAYERS))

        pending_sends = []
        acc = x_ref[...].astype(jnp.bfloat16)
        for layer in range(N_LAYERS):
            h = jnp.dot(acc, wins[layer][...],
                        preferred_element_type=jnp.float32)
            h = jnp.maximum(h, 0.0).astype(jnp.bfloat16)
            acc = jnp.dot(h, wouts[layer][...],
                          preferred_element_type=jnp.float32
                          ).astype(jnp.bfloat16)
            slot = layer * N_RECV
            for si, group in enumerate(GROUPS):
                send_slot = layer * N_STAGES + si
                send_ref[send_slot] = acc
                started = []
                for g in group:
                    rdma = pltpu.make_async_remote_copy(
                        src_ref=send_ref.at[send_slot],
                        dst_ref=recv_ref.at[slot],
                        send_sem=send_sems.at[slot],
                        recv_sem=recv_sems.at[slot],
                        device_id=(my ^ g,),
                        device_id_type=pl.DeviceIdType.MESH,
                    )
                    rdma.start()
                    started.append((rdma, slot))
                    slot += 1
                pending_sends.extend(r for r, _ in started)
                for rdma, rslot in started:
                    rdma.wait_recv()
                    acc = acc + recv_ref[rslot]

        out_ref[...] = acc.astype(jnp.float32)
        for rdma in pending_sends:
            rdma.wait_send()

    bf = jnp.bfloat16
    h_per = Win0.shape[1]
    return pl.pallas_call(
        body,
        out_shape=jax.ShapeDtypeStruct((b, d), jnp.float32),
        in_specs=[pl.BlockSpec(memory_space=pltpu.VMEM)] * 7,
        out_specs=pl.BlockSpec(memory_space=pltpu.VMEM),
        scratch_shapes=[
            pltpu.VMEM((N_LAYERS, b, h_per), bf),
            pltpu.VMEM((N_LAYERS, h_per, d), bf),
            pltpu.VMEM((N_LAYERS * N_STAGES, b, d), bf),
            pltpu.VMEM((N_SLOTS, b, d), bf),
            pltpu.SemaphoreType.DMA((N_SLOTS,)),
            pltpu.SemaphoreType.DMA((N_SLOTS,)),
        ],
        compiler_params=pltpu.CompilerParams(collective_id=0),
    )(x, Win0, Wout0, Win1, Wout1, Win2, Wout2)


# device time: 37396 ns/iter; 1.0858x vs baseline; 1.0858x over previous
import jax
import jax.numpy as jnp
from jax import lax
from jax.experimental import pallas as pl
from jax.experimental.pallas import tpu as pltpu

N_DEV = 32
N_LAYERS = 3
GROUPS = ((1, 3, 2, 4, 5, 7, 6), (8, 16, 24))
ALL_GENS = tuple(g for grp in GROUPS for g in grp)
N_STAGES = len(GROUPS)
N_RECV = len(ALL_GENS)
N_SLOTS = N_LAYERS * N_RECV


def kernel(x, Win0, Wout0, Win1, Wout1, Win2, Wout2):
    b, d = x.shape

    def body(x_ref, wins_ref, wouts_ref, out_ref, send_ref, recv_ref,
             send_sems, recv_sems):
        my = lax.axis_index("i")

        barrier_sem = pltpu.get_barrier_semaphore()
        for g in ALL_GENS:
            pl.semaphore_signal(
                barrier_sem, inc=1,
                device_id=(my ^ g,), device_id_type=pl.DeviceIdType.MESH,
            )
        pl.semaphore_wait(barrier_sem, N_RECV)

        pending_sends = []
        acc = x_ref[...]
        for layer in range(N_LAYERS):
            h = jnp.dot(acc, wins_ref[layer],
                        preferred_element_type=jnp.float32)
            h = jnp.maximum(h, 0.0).astype(jnp.bfloat16)
            acc = jnp.dot(h, wouts_ref[layer],
                          preferred_element_type=jnp.float32
                          ).astype(jnp.bfloat16)
            slot = layer * N_RECV
            for si, group in enumerate(GROUPS):
                send_slot = layer * N_STAGES + si
                send_ref[send_slot] = acc
                started = []
                for g in group:
                    rdma = pltpu.make_async_remote_copy(
                        src_ref=send_ref.at[send_slot],
                        dst_ref=recv_ref.at[slot],
                        send_sem=send_sems.at[slot],
                        recv_sem=recv_sems.at[slot],
                        device_id=(my ^ g,),
                        device_id_type=pl.DeviceIdType.MESH,
                    )
                    rdma.start()
                    started.append((rdma, slot))
                    slot += 1
                pending_sends.extend(r for r, _ in started)
                for rdma, rslot in started:
                    rdma.wait_recv()
                    acc = acc + recv_ref[rslot]

        out_ref[...] = acc.astype(jnp.float32)
        for rdma in pending_sends:
            rdma.wait_send()

    bf = jnp.bfloat16
    wins = jnp.stack([Win0.astype(bf), Win1.astype(bf), Win2.astype(bf)])
    wouts = jnp.stack([Wout0.astype(bf), Wout1.astype(bf), Wout2.astype(bf)])
    return pl.pallas_call(
        body,
        out_shape=jax.ShapeDtypeStruct((b, d), jnp.float32),
        in_specs=[pl.BlockSpec(memory_space=pltpu.VMEM)] * 3,
        out_specs=pl.BlockSpec(memory_space=pltpu.VMEM),
        scratch_shapes=[
            pltpu.VMEM((N_LAYERS * N_STAGES, b, d), bf),
            pltpu.VMEM((N_SLOTS, b, d), bf),
            pltpu.SemaphoreType.DMA((N_SLOTS,)),
            pltpu.SemaphoreType.DMA((N_SLOTS,)),
        ],
        compiler_params=pltpu.CompilerParams(collective_id=0),
    )(x.astype(bf), wins, wouts)


# device time: 35209 ns/iter; 1.1532x vs baseline; 1.0621x over previous
import jax
import jax.numpy as jnp
from jax import lax
from jax.experimental import pallas as pl
from jax.experimental.pallas import tpu as pltpu

N_DEV = 32
N_LAYERS = 3
PLANE = (1, 3, 2, 4, 5, 7, 6)
ZAXIS = (8, 16, 24)
ALL_GENS = PLANE + ZAXIS
N_RECV = len(ALL_GENS)
N_RECV_SLOTS = 2 * N_RECV
N_SLOTS = N_LAYERS * N_RECV_SLOTS
N_SEND = 4


def kernel(x, Win0, Wout0, Win1, Wout1, Win2, Wout2):
    b, d = x.shape

    def body(x_ref, wins_ref, wouts_ref, out_ref, send_ref, recv_ref,
             send_sems, recv_sems):
        my = lax.axis_index("i")

        barrier_sem = pltpu.get_barrier_semaphore()
        for g in ALL_GENS:
            pl.semaphore_signal(
                barrier_sem, inc=1,
                device_id=(my ^ g,), device_id_type=pl.DeviceIdType.MESH,
            )
        pl.semaphore_wait(barrier_sem, N_RECV)

        pending_sends = []

        def exchange(val, send_slot, gens, slot0):
            send_ref[send_slot] = val
            started = []
            slot = slot0
            for g in gens:
                rdma = pltpu.make_async_remote_copy(
                    src_ref=send_ref.at[send_slot],
                    dst_ref=recv_ref.at[slot],
                    send_sem=send_sems.at[slot],
                    recv_sem=recv_sems.at[slot],
                    device_id=(my ^ g,),
                    device_id_type=pl.DeviceIdType.MESH,
                )
                rdma.start()
                started.append((rdma, slot))
                slot += 1
            pending_sends.extend(r for r, _ in started)
            return started, slot

        def collect(val, started):
            for rdma, rslot in started:
                rdma.wait_recv()
                val = val + recv_ref[rslot]
            return val

        acc = x_ref[...]
        hb = b // 2
        for layer in range(N_LAYERS):
            h = jnp.dot(acc, wins_ref[layer],
                        preferred_element_type=jnp.float32)
            h = jnp.maximum(h, 0.0).astype(jnp.bfloat16)
            acc = jnp.dot(h, wouts_ref[layer],
                          preferred_element_type=jnp.float32
                          ).astype(jnp.bfloat16)
            slot = layer * N_RECV_SLOTS
            sbase = layer * N_SEND
            a_half = acc[:hb, :]
            b_half = acc[hb:, :]
            a1, slot = exchange(a_half, sbase + 0, PLANE, slot)
            b1, slot = exchange(b_half, sbase + 1, ZAXIS, slot)
            a_sum = collect(a_half, a1)
            a2, slot = exchange(a_sum, sbase + 2, ZAXIS, slot)
            b_sum = collect(b_half, b1)
            b2, slot = exchange(b_sum, sbase + 3, PLANE, slot)
            a_fin = collect(a_sum, a2)
            b_fin = collect(b_sum, b2)
            acc = jnp.concatenate([a_fin, b_fin], axis=0)

        out_ref[...] = acc.astype(jnp.float32)
        for rdma in pending_sends:
            rdma.wait_send()

    bf = jnp.bfloat16
    wins = jnp.stack([Win0.astype(bf), Win1.astype(bf), Win2.astype(bf)])
    wouts = jnp.stack([Wout0.astype(bf), Wout1.astype(bf), Wout2.astype(bf)])
    return pl.pallas_call(
        body,
        out_shape=jax.ShapeDtypeStruct((b, d), jnp.float32),
        in_specs=[pl.BlockSpec(memory_space=pltpu.VMEM)] * 3,
        out_specs=pl.BlockSpec(memory_space=pltpu.VMEM),
        scratch_shapes=[
            pltpu.VMEM((N_LAYERS * N_SEND, b // 2, d), bf),
            pltpu.VMEM((N_SLOTS, b // 2, d), bf),
            pltpu.SemaphoreType.DMA((N_SLOTS,)),
            pltpu.SemaphoreType.DMA((N_SLOTS,)),
        ],
        compiler_params=pltpu.CompilerParams(collective_id=0),
    )(x.astype(bf), wins, wouts)


# device time: 33527 ns/iter; 1.2111x vs baseline; 1.0502x over previous
import jax
import jax.numpy as jnp
from jax import lax
from jax.experimental import pallas as pl
from jax.experimental.pallas import tpu as pltpu

N_DEV = 32
N_LAYERS = 3
PLANE = (1, 3, 2, 4, 5, 7, 6)
ZAXIS = (8, 16, 24)
ALL_GENS = PLANE + ZAXIS
N_RECV = len(ALL_GENS)
N_RECV_SLOTS = 2 * N_RECV
N_SLOTS = N_LAYERS * N_RECV_SLOTS
N_SEND = 4


def kernel(x, Win0, Wout0, Win1, Wout1, Win2, Wout2):
    b, d = x.shape

    def body(x_ref, wins_ref, wouts_ref, out_ref, send_ref, recv_ref,
             send_sems, recv_sems):
        my = lax.axis_index("i")

        barrier_sem = pltpu.get_barrier_semaphore()
        for g in ALL_GENS:
            pl.semaphore_signal(
                barrier_sem, inc=1,
                device_id=(my ^ g,), device_id_type=pl.DeviceIdType.MESH,
            )
        pl.semaphore_wait(barrier_sem, N_RECV)

        pending_sends = []

        def exchange(val, send_slot, gens, slot0):
            send_ref[send_slot] = val
            started = []
            slot = slot0
            for g in gens:
                rdma = pltpu.make_async_remote_copy(
                    src_ref=send_ref.at[send_slot],
                    dst_ref=recv_ref.at[slot],
                    send_sem=send_sems.at[slot],
                    recv_sem=recv_sems.at[slot],
                    device_id=(my ^ g,),
                    device_id_type=pl.DeviceIdType.MESH,
                )
                rdma.start()
                started.append((rdma, slot))
                slot += 1
            pending_sends.extend(r for r, _ in started)
            return started, slot

        def collect(val, started):
            for rdma, rslot in started:
                rdma.wait_recv()
                val = val + recv_ref[rslot]
            return val

        def mlp_half(val, layer):
            h = jnp.dot(val, wins_ref[layer],
                        preferred_element_type=jnp.float32)
            h = jnp.maximum(h, 0.0).astype(jnp.bfloat16)
            return jnp.dot(h, wouts_ref[layer],
                           preferred_element_type=jnp.float32
                           ).astype(jnp.bfloat16)

        hb = b // 2
        a_val = x_ref[:hb, :]
        b_prev = (x_ref[hb:, :], None)
        for layer in range(N_LAYERS):
            slot = layer * N_RECV_SLOTS
            sbase = layer * N_SEND
            a_part = mlp_half(a_val, layer)
            a1, slot = exchange(a_part, sbase + 0, PLANE, slot)
            b_base, b_pend = b_prev
            b_fin = collect(b_base, b_pend) if b_pend else b_base
            b_part = mlp_half(b_fin, layer)
            b1, slot = exchange(b_part, sbase + 1, ZAXIS, slot)
            a_sum = collect(a_part, a1)
            a2, slot = exchange(a_sum, sbase + 2, ZAXIS, slot)
            b_sum = collect(b_part, b1)
            b2, slot = exchange(b_sum, sbase + 3, PLANE, slot)
            a_val = collect(a_sum, a2)
            b_prev = (b_sum, b2)

        out_ref[:hb, :] = a_val.astype(jnp.float32)
        out_ref[hb:, :] = collect(b_prev[0], b_prev[1]).astype(jnp.float32)
        for rdma in pending_sends:
            rdma.wait_send()

    bf = jnp.bfloat16
    wins = jnp.stack([Win0.astype(bf), Win1.astype(bf), Win2.astype(bf)])
    wouts = jnp.stack([Wout0.astype(bf), Wout1.astype(bf), Wout2.astype(bf)])
    return pl.pallas_call(
        body,
        out_shape=jax.ShapeDtypeStruct((b, d), jnp.float32),
        in_specs=[pl.BlockSpec(memory_space=pltpu.VMEM)] * 3,
        out_specs=pl.BlockSpec(memory_space=pltpu.VMEM),
        scratch_shapes=[
            pltpu.VMEM((N_LAYERS * N_SEND, b // 2, d), bf),
            pltpu.VMEM((N_SLOTS, b // 2, d), bf),
            pltpu.SemaphoreType.DMA((N_SLOTS,)),
            pltpu.SemaphoreType.DMA((N_SLOTS,)),
        ],
        compiler_params=pltpu.CompilerParams(collective_id=0),
    )(x.astype(bf), wins, wouts)
